# baseline (device time: 157764 ns/iter reference)
import jax
import jax.numpy as jnp
from jax import lax
from jax.experimental import pallas as pl
from jax.experimental.pallas import tpu as pltpu

N_DEV = 32
E_PER = 2
N_EXP = 64
T_LOC = 256
D = 128
H = 256
CAP = 102


def _barrier(my):
    sem = pltpu.get_barrier_semaphore()
    for p in range(1, N_DEV):
        pl.semaphore_signal(
            sem, inc=1,
            device_id=(lax.rem(my + p, N_DEV),),
            device_id_type=pl.DeviceIdType.MESH,
        )
    pl.semaphore_wait(sem, N_DEV - 1)


def _gather_ridx(ridx_row):

    def body(ridx_ref, out_ref, send_ri, recv_ri):
        my = lax.axis_index("i")
        _barrier(my)
        out_ref[pl.ds(my, 1), :] = ridx_ref[...]
        sends = []
        for d in range(1, N_DEV):
            tgt = lax.rem(my + d, N_DEV)
            r = pltpu.make_async_remote_copy(
                src_ref=ridx_ref,
                dst_ref=out_ref.at[pl.ds(my, 1), :],
                send_sem=send_ri.at[d - 1],
                recv_sem=recv_ri.at[my],
                device_id=(tgt,),
                device_id_type=pl.DeviceIdType.MESH,
            )
            r.start()
            sends.append(r)
        for d in range(1, N_DEV):
            src = lax.rem(my - d + N_DEV, N_DEV)
            pltpu.make_async_remote_copy(
                src_ref=ridx_ref,
                dst_ref=out_ref.at[pl.ds(src, 1), :],
                send_sem=send_ri.at[0],
                recv_sem=recv_ri.at[src],
                device_id=(src,),
                device_id_type=pl.DeviceIdType.MESH,
            ).wait_recv()
        for r in sends:
            r.wait_send()

    return pl.pallas_call(
        body,
        out_shape=jax.ShapeDtypeStruct((N_DEV, T_LOC), jnp.int32),
        in_specs=[pl.BlockSpec(memory_space=pltpu.VMEM)],
        out_specs=pl.BlockSpec(memory_space=pltpu.VMEM),
        scratch_shapes=[
            pltpu.SemaphoreType.DMA((N_DEV - 1,)),
            pltpu.SemaphoreType.DMA((N_DEV,)),
        ],
        compiler_params=pltpu.CompilerParams(collective_id=0),
    )(ridx_row)


def kernel(x, router_W, route_idx, expert_W):
    del router_W
    x = x.astype(jnp.bfloat16).reshape(T_LOC, 1, D)
    ew = expert_W.astype(jnp.bfloat16)
    ridx_row = route_idx.reshape(1, T_LOC)

    me = lax.axis_index("i")

    ridx_all = _gather_ridx(ridx_row)

    flat = ridx_all.reshape(N_DEV * T_LOC)
    oh = (flat[:, None] == jnp.arange(N_EXP)[None, :]).astype(jnp.int32)
    excl = jnp.cumsum(oh, axis=0) - oh
    rank_full = jnp.sum(excl * oh, axis=1)
    keep_full = (rank_full < CAP).astype(jnp.int32)

    e_loc = lax.dynamic_slice(flat, (me * T_LOC,), (T_LOC,))
    rank_loc = lax.dynamic_slice(rank_full, (me * T_LOC,), (T_LOC,))
    keep_t = lax.dynamic_slice(keep_full, (me * T_LOC,), (T_LOC,))
    dst_t = (e_loc // E_PER).astype(jnp.int32)
    slot_t = ((e_loc % E_PER) * CAP
              + jnp.minimum(rank_loc, CAP - 1)).astype(jnp.int32)

    counts = jnp.sum(oh, axis=0)
    n_t = jnp.minimum(
        lax.dynamic_slice(counts, (me * E_PER,), (E_PER,)), CAP
    ).astype(jnp.int32)

    gs = [jnp.nonzero(flat == me * E_PER + k, size=CAP, fill_value=0)[0]
          for k in range(E_PER)]
    t_g = jnp.stack(gs).astype(jnp.int32)
    td_t = t_g // T_LOC
    tr_t = t_g % T_LOC

    def body(keep_ref, dst_ref, slot_ref, n_ref, td_ref, tr_ref,
             x_ref, ew_ref, out_ref,
             x_stage, res_ref, send_x, recv_x, send_r, recv_r):
        my = lax.axis_index("i")
        _barrier(my)
        out_ref[...] = jnp.zeros((T_LOC, 1, H), jnp.float32)

        def disp(i, c):
            @pl.when((keep_ref[i] != 0) & (dst_ref[i] != my))
            def _():
                pltpu.make_async_remote_copy(
                    src_ref=x_ref.at[i],
                    dst_ref=x_stage.at[slot_ref[i]],
                    send_sem=send_x.at[i],
                    recv_sem=recv_x.at[slot_ref[i]],
                    device_id=(dst_ref[i],),
                    device_id_type=pl.DeviceIdType.MESH,
                ).start()

            @pl.when((keep_ref[i] != 0) & (dst_ref[i] == my))
            def _():
                x_stage[slot_ref[i]] = x_ref[i]
            return c
        lax.fori_loop(0, T_LOC, disp, 0)

        for k in range(E_PER):
            def wrecv(r, c, k=k):
                s = k * CAP + r

                @pl.when(td_ref[k, r] != my)
                def _():
                    pltpu.make_async_remote_copy(
                        src_ref=x_ref.at[0],
                        dst_ref=x_stage.at[s],
                        send_sem=send_x.at[0],
                        recv_sem=recv_x.at[s],
                        device_id=(my,),
                        device_id_type=pl.DeviceIdType.MESH,
                    ).wait_recv()
                return c
            lax.fori_loop(0, n_ref[k], wrecv, 0)

        iota = lax.broadcasted_iota(jnp.int32, (CAP, 1), 0)
        for k in range(E_PER):
            mask = (iota < n_ref[k]).astype(jnp.bfloat16)
            xs = x_stage[k * CAP:(k + 1) * CAP].reshape(CAP, D) * mask
            y = jnp.dot(xs, ew_ref[k], preferred_element_type=jnp.float32)
            res_ref[k * CAP:(k + 1) * CAP] = y.reshape(CAP, 1, H)

        rets = []
        for k in range(E_PER):
            def ret(r, c, k=k):
                s = k * CAP + r

                @pl.when(td_ref[k, r] != my)
                def _():
                    pltpu.make_async_remote_copy(
                        src_ref=res_ref.at[s],
                        dst_ref=out_ref.at[tr_ref[k, r]],
                        send_sem=send_r.at[s],
                        recv_sem=recv_r.at[tr_ref[k, r]],
                        device_id=(td_ref[k, r],),
                        device_id_type=pl.DeviceIdType.MESH,
                    ).start()

                @pl.when(td_ref[k, r] == my)
                def _():
                    out_ref[tr_ref[k, r]] = res_ref[s]
                return c
            lax.fori_loop(0, n_ref[k], ret, 0)

        def coll(i, c):
            @pl.when((keep_ref[i] != 0) & (dst_ref[i] != my))
            def _():
                pltpu.make_async_remote_copy(
                    src_ref=res_ref.at[0],
                    dst_ref=out_ref.at[i],
                    send_sem=send_r.at[0],
                    recv_sem=recv_r.at[i],
                    device_id=(my,),
                    device_id_type=pl.DeviceIdType.MESH,
                ).wait_recv()
            return c
        lax.fori_loop(0, T_LOC, coll, 0)

        def dsend(i, c):
            @pl.when((keep_ref[i] != 0) & (dst_ref[i] != my))
            def _():
                pltpu.make_async_remote_copy(
                    src_ref=x_ref.at[i],
                    dst_ref=x_stage.at[0],
                    send_sem=send_x.at[i],
                    recv_sem=recv_x.at[0],
                    device_id=(my,),
                    device_id_type=pl.DeviceIdType.MESH,
                ).wait_send()
            return c
        lax.fori_loop(0, T_LOC, dsend, 0)
        for k in range(E_PER):
            def dret(r, c, k=k):
                s = k * CAP + r

                @pl.when(td_ref[k, r] != my)
                def _():
                    pltpu.make_async_remote_copy(
                        src_ref=res_ref.at[s],
                        dst_ref=out_ref.at[0],
                        send_sem=send_r.at[s],
                        recv_sem=recv_r.at[0],
                        device_id=(my,),
                        device_id_type=pl.DeviceIdType.MESH,
                    ).wait_send()
                return c
            lax.fori_loop(0, n_ref[k], dret, 0)

    smem = pl.BlockSpec(memory_space=pltpu.SMEM)
    vmem = pl.BlockSpec(memory_space=pltpu.VMEM)
    out = pl.pallas_call(
        body,
        out_shape=jax.ShapeDtypeStruct((T_LOC, 1, H), jnp.float32),
        in_specs=[smem, smem, smem, smem, smem, smem, vmem, vmem],
        out_specs=vmem,
        scratch_shapes=[
            pltpu.VMEM((E_PER * CAP, 1, D), jnp.bfloat16),
            pltpu.VMEM((E_PER * CAP, 1, H), jnp.float32),
            pltpu.SemaphoreType.DMA((T_LOC,)),
            pltpu.SemaphoreType.DMA((E_PER * CAP,)),
            pltpu.SemaphoreType.DMA((E_PER * CAP,)),
            pltpu.SemaphoreType.DMA((T_LOC,)),
        ],
        compiler_params=pltpu.CompilerParams(collective_id=1),
    )(keep_t, dst_t, slot_t, n_t, td_t, tr_t, x, ew)
    return out.reshape(T_LOC, H)


# device time: 64192 ns/iter; 2.4577x vs baseline; 2.4577x over previous
import jax
import jax.numpy as jnp
from jax import lax
from jax.experimental import pallas as pl
from jax.experimental.pallas import tpu as pltpu

N_DEV = 32
E_PER = 2
N_EXP = 64
T_LOC = 256
D = 128
H = 256
CAP = 102


def _barrier(my):
    sem = pltpu.get_barrier_semaphore()
    for p in range(1, N_DEV):
        pl.semaphore_signal(
            sem, inc=1,
            device_id=(lax.rem(my + p, N_DEV),),
            device_id_type=pl.DeviceIdType.MESH,
        )
    pl.semaphore_wait(sem, N_DEV - 1)


def _route_tables(ridx_row, ridx_col):

    def body(ridx_ref, ridx_col_ref,
             keep_out, dst_out, slot_out, n_out, td_out, tr_out,
             gbuf, send_ri, recv_ri):
        my = lax.axis_index("i")
        _barrier(my)
        gbuf[pl.ds(my, 1), :] = ridx_ref[...]
        sends = []
        for d in range(1, N_DEV):
            tgt = lax.rem(my + d, N_DEV)
            r = pltpu.make_async_remote_copy(
                src_ref=ridx_ref,
                dst_ref=gbuf.at[pl.ds(my, 1), :],
                send_sem=send_ri.at[d - 1],
                recv_sem=recv_ri.at[my],
                device_id=(tgt,),
                device_id_type=pl.DeviceIdType.MESH,
            )
            r.start()
            sends.append(r)
        for d in range(1, N_DEV):
            src = lax.rem(my - d + N_DEV, N_DEV)
            pltpu.make_async_remote_copy(
                src_ref=ridx_ref,
                dst_ref=gbuf.at[pl.ds(src, 1), :],
                send_sem=send_ri.at[0],
                recv_sem=recv_ri.at[src],
                device_id=(src,),
                device_id_type=pl.DeviceIdType.MESH,
            ).wait_recv()

        ridx_all = gbuf[...]
        iota_e3 = lax.broadcasted_iota(jnp.int32, (N_DEV, T_LOC, N_EXP), 2)
        onehot3 = (ridx_all[:, :, None] == iota_e3).astype(jnp.float32)
        shard_iota = lax.broadcasted_iota(jnp.int32, (N_DEV, 1, 1), 0)
        before = (shard_iota < my).astype(jnp.float32)
        base = jnp.sum(onehot3 * before, axis=(0, 1))

        e_col = ridx_col_ref[...]
        iota_lane = lax.broadcasted_iota(jnp.int32, (T_LOC, N_EXP), 1)
        onehot_loc = (e_col == iota_lane).astype(jnp.float32)
        ii = lax.broadcasted_iota(jnp.int32, (T_LOC, T_LOC), 0)
        jj = lax.broadcasted_iota(jnp.int32, (T_LOC, T_LOC), 1)
        ltri = (jj < ii).astype(jnp.float32)
        excl = jnp.dot(ltri, onehot_loc,
                       preferred_element_type=jnp.float32)
        rank = jnp.sum(onehot_loc * (base[None, :] + excl),
                       axis=1, keepdims=True)
        keep_out[...] = (rank < CAP).astype(jnp.int32)
        dst_out[...] = e_col // E_PER
        slot_out[...] = ((e_col % E_PER) * CAP
                         + jnp.minimum(rank.astype(jnp.int32), CAP - 1))

        tot_row = jnp.sum(jnp.sum(onehot3, axis=0), axis=0,
                          keepdims=True)
        iota_k = lax.broadcasted_iota(jnp.int32, (E_PER, N_EXP), 0)
        iota_e2 = lax.broadcasted_iota(jnp.int32, (E_PER, N_EXP), 1)
        sel = (iota_e2 == E_PER * my + iota_k).astype(jnp.float32)
        n_pair = jnp.sum(sel * tot_row, axis=1, keepdims=True)
        n_out[...] = jnp.minimum(n_pair, float(CAP)).astype(jnp.int32)

        iota_s = lax.broadcasted_iota(jnp.int32, (N_DEV, T_LOC), 0)
        iota_j = lax.broadcasted_iota(jnp.int32, (N_DEV, T_LOC), 1)
        gval = (iota_s * T_LOC + iota_j).astype(jnp.float32)
        m_up = (ii < jj).astype(jnp.float32)
        l32a = lax.broadcasted_iota(jnp.int32, (N_DEV, N_DEV), 0)
        l32b = lax.broadcasted_iota(jnp.int32, (N_DEV, N_DEV), 1)
        l32 = (l32b < l32a).astype(jnp.float32)
        iota_r = lax.broadcasted_iota(jnp.int32, (N_DEV, T_LOC, CAP), 2)
        tds, trs = [], []
        for k in range(E_PER):
            maskk = (ridx_all == E_PER * my + k).astype(jnp.float32)
            cntk = jnp.sum(maskk, axis=1, keepdims=True)
            pk = jnp.dot(l32, cntk, preferred_element_type=jnp.float32)
            lexcl = jnp.dot(maskk, m_up,
                            preferred_element_type=jnp.float32)
            rankk = (pk + lexcl).astype(jnp.int32)
            validk = maskk * (rankk < CAP).astype(jnp.float32)
            oh_r = (rankk[:, :, None] == iota_r).astype(jnp.float32)
            contrib = (validk * gval)[:, :, None] * oh_r
            tk = jnp.sum(jnp.sum(contrib, axis=0), axis=0,
                         keepdims=True).astype(jnp.int32)
            tds.append(tk // T_LOC)
            trs.append(tk % T_LOC)
        td_out[...] = jnp.concatenate(tds, axis=0)
        tr_out[...] = jnp.concatenate(trs, axis=0)

        for r in sends:
            r.wait_send()

    i32 = jnp.int32
    return pl.pallas_call(
        body,
        out_shape=[
            jax.ShapeDtypeStruct((T_LOC, 1), i32),
            jax.ShapeDtypeStruct((T_LOC, 1), i32),
            jax.ShapeDtypeStruct((T_LOC, 1), i32),
            jax.ShapeDtypeStruct((E_PER, 1), i32),
            jax.ShapeDtypeStruct((E_PER, CAP), i32),
            jax.ShapeDtypeStruct((E_PER, CAP), i32),
        ],
        in_specs=[pl.BlockSpec(memory_space=pltpu.VMEM)] * 2,
        out_specs=[pl.BlockSpec(memory_space=pltpu.VMEM)] * 6,
        scratch_shapes=[
            pltpu.VMEM((N_DEV, T_LOC), jnp.int32),
            pltpu.SemaphoreType.DMA((N_DEV - 1,)),
            pltpu.SemaphoreType.DMA((N_DEV,)),
        ],
        compiler_params=pltpu.CompilerParams(collective_id=0),
    )(ridx_row, ridx_col)


def kernel(x, router_W, route_idx, expert_W):
    del router_W
    x = x.astype(jnp.bfloat16).reshape(T_LOC, 1, D)
    ew = expert_W.astype(jnp.bfloat16)

    keep_t, dst_t, slot_t, n_t, td_t, tr_t = _route_tables(
        route_idx.reshape(1, T_LOC), route_idx.reshape(T_LOC, 1))
    keep_t = keep_t.reshape(T_LOC)
    dst_t = dst_t.reshape(T_LOC)
    slot_t = slot_t.reshape(T_LOC)
    n_t = n_t.reshape(E_PER)

    def body(keep_ref, dst_ref, slot_ref, n_ref, td_ref, tr_ref,
             x_ref, ew_ref, out_ref,
             x_stage, res_ref, send_x, recv_x, send_r, recv_r):
        my = lax.axis_index("i")
        _barrier(my)
        out_ref[...] = jnp.zeros((T_LOC, 1, H), jnp.float32)

        def disp(i, c):
            @pl.when((keep_ref[i] != 0) & (dst_ref[i] != my))
            def _():
                pltpu.make_async_remote_copy(
                    src_ref=x_ref.at[i],
                    dst_ref=x_stage.at[slot_ref[i]],
                    send_sem=send_x.at[i],
                    recv_sem=recv_x.at[slot_ref[i]],
                    device_id=(dst_ref[i],),
                    device_id_type=pl.DeviceIdType.MESH,
                ).start()

            @pl.when((keep_ref[i] != 0) & (dst_ref[i] == my))
            def _():
                x_stage[slot_ref[i]] = x_ref[i]
            return c
        lax.fori_loop(0, T_LOC, disp, 0)

        for k in range(E_PER):
            def wrecv(r, c, k=k):
                s = k * CAP + r

                @pl.when(td_ref[k, r] != my)
                def _():
                    pltpu.make_async_remote_copy(
                        src_ref=x_ref.at[0],
                        dst_ref=x_stage.at[s],
                        send_sem=send_x.at[0],
                        recv_sem=recv_x.at[s],
                        device_id=(my,),
                        device_id_type=pl.DeviceIdType.MESH,
                    ).wait_recv()
                return c
            lax.fori_loop(0, n_ref[k], wrecv, 0)

        iota = lax.broadcasted_iota(jnp.int32, (CAP, 1), 0)
        for k in range(E_PER):
            mask = (iota < n_ref[k]).astype(jnp.bfloat16)
            xs = x_stage[k * CAP:(k + 1) * CAP].reshape(CAP, D) * mask
            y = jnp.dot(xs, ew_ref[k], preferred_element_type=jnp.float32)
            res_ref[k * CAP:(k + 1) * CAP] = y.reshape(CAP, 1, H)

        for k in range(E_PER):
            def ret(r, c, k=k):
                s = k * CAP + r

                @pl.when(td_ref[k, r] != my)
                def _():
                    pltpu.make_async_remote_copy(
                        src_ref=res_ref.at[s],
                        dst_ref=out_ref.at[tr_ref[k, r]],
                        send_sem=send_r.at[s],
                        recv_sem=recv_r.at[tr_ref[k, r]],
                        device_id=(td_ref[k, r],),
                        device_id_type=pl.DeviceIdType.MESH,
                    ).start()

                @pl.when(td_ref[k, r] == my)
                def _():
                    out_ref[tr_ref[k, r]] = res_ref[s]
                return c
            lax.fori_loop(0, n_ref[k], ret, 0)

        def coll(i, c):
            @pl.when((keep_ref[i] != 0) & (dst_ref[i] != my))
            def _():
                pltpu.make_async_remote_copy(
                    src_ref=res_ref.at[0],
                    dst_ref=out_ref.at[i],
                    send_sem=send_r.at[0],
                    recv_sem=recv_r.at[i],
                    device_id=(my,),
                    device_id_type=pl.DeviceIdType.MESH,
                ).wait_recv()
            return c
        lax.fori_loop(0, T_LOC, coll, 0)

        def dsend(i, c):
            @pl.when((keep_ref[i] != 0) & (dst_ref[i] != my))
            def _():
                pltpu.make_async_remote_copy(
                    src_ref=x_ref.at[i],
                    dst_ref=x_stage.at[0],
                    send_sem=send_x.at[i],
                    recv_sem=recv_x.at[0],
                    device_id=(my,),
                    device_id_type=pl.DeviceIdType.MESH,
                ).wait_send()
            return c
        lax.fori_loop(0, T_LOC, dsend, 0)
        for k in range(E_PER):
            def dret(r, c, k=k):
                s = k * CAP + r

                @pl.when(td_ref[k, r] != my)
                def _():
                    pltpu.make_async_remote_copy(
                        src_ref=res_ref.at[s],
                        dst_ref=out_ref.at[0],
                        send_sem=send_r.at[s],
                        recv_sem=recv_r.at[0],
                        device_id=(my,),
                        device_id_type=pl.DeviceIdType.MESH,
                    ).wait_send()
                return c
            lax.fori_loop(0, n_ref[k], dret, 0)

    smem = pl.BlockSpec(memory_space=pltpu.SMEM)
    vmem = pl.BlockSpec(memory_space=pltpu.VMEM)
    out = pl.pallas_call(
        body,
        out_shape=jax.ShapeDtypeStruct((T_LOC, 1, H), jnp.float32),
        in_specs=[smem, smem, smem, smem, smem, smem, vmem, vmem],
        out_specs=vmem,
        scratch_shapes=[
            pltpu.VMEM((E_PER * CAP, 1, D), jnp.bfloat16),
            pltpu.VMEM((E_PER * CAP, 1, H), jnp.float32),
            pltpu.SemaphoreType.DMA((T_LOC,)),
            pltpu.SemaphoreType.DMA((E_PER * CAP,)),
            pltpu.SemaphoreType.DMA((E_PER * CAP,)),
            pltpu.SemaphoreType.DMA((T_LOC,)),
        ],
        compiler_params=pltpu.CompilerParams(collective_id=1),
    )(keep_t, dst_t, slot_t, n_t, td_t, tr_t, x, ew)
    return out.reshape(T_LOC, H)


# device time: 62641 ns/iter; 2.5185x vs baseline; 1.0248x over previous
import jax
import jax.numpy as jnp
from jax import lax
from jax.experimental import pallas as pl
from jax.experimental.pallas import tpu as pltpu

N_DEV = 32
E_PER = 2
N_EXP = 64
T_LOC = 256
D = 128
H = 256
CAP = 102


def _barrier(my):
    sem = pltpu.get_barrier_semaphore()
    for p in range(1, N_DEV):
        pl.semaphore_signal(
            sem, inc=1,
            device_id=(lax.rem(my + p, N_DEV),),
            device_id_type=pl.DeviceIdType.MESH,
        )
    pl.semaphore_wait(sem, N_DEV - 1)


def _route_tables(ridx_row, ridx_col):

    def body(ridx_ref, ridx_col_ref,
             keep_out, dst_out, slot_out, n_out, td_out, tr_out,
             gbuf, send_ri, recv_ri):
        my = lax.axis_index("i")
        _barrier(my)
        gbuf[pl.ds(my, 1), :] = ridx_ref[...]
        sends = []
        for d in range(1, N_DEV):
            tgt = lax.rem(my + d, N_DEV)
            r = pltpu.make_async_remote_copy(
                src_ref=ridx_ref,
                dst_ref=gbuf.at[pl.ds(my, 1), :],
                send_sem=send_ri.at[d - 1],
                recv_sem=recv_ri.at[my],
                device_id=(tgt,),
                device_id_type=pl.DeviceIdType.MESH,
            )
            r.start()
            sends.append(r)
        for d in range(1, N_DEV):
            src = lax.rem(my - d + N_DEV, N_DEV)
            pltpu.make_async_remote_copy(
                src_ref=ridx_ref,
                dst_ref=gbuf.at[pl.ds(src, 1), :],
                send_sem=send_ri.at[0],
                recv_sem=recv_ri.at[src],
                device_id=(src,),
                device_id_type=pl.DeviceIdType.MESH,
            ).wait_recv()

        ridx_all = gbuf[...]
        iota_e3 = lax.broadcasted_iota(jnp.int32, (N_DEV, T_LOC, N_EXP), 2)
        onehot3 = (ridx_all[:, :, None] == iota_e3).astype(jnp.float32)
        shard_iota = lax.broadcasted_iota(jnp.int32, (N_DEV, 1, 1), 0)
        before = (shard_iota < my).astype(jnp.float32)
        base = jnp.sum(onehot3 * before, axis=(0, 1))

        e_col = ridx_col_ref[...]
        iota_lane = lax.broadcasted_iota(jnp.int32, (T_LOC, N_EXP), 1)
        onehot_loc = (e_col == iota_lane).astype(jnp.float32)
        ii = lax.broadcasted_iota(jnp.int32, (T_LOC, T_LOC), 0)
        jj = lax.broadcasted_iota(jnp.int32, (T_LOC, T_LOC), 1)
        ltri = (jj < ii).astype(jnp.float32)
        excl = jnp.dot(ltri, onehot_loc,
                       preferred_element_type=jnp.float32)
        rank = jnp.sum(onehot_loc * (base[None, :] + excl),
                       axis=1, keepdims=True)
        keep_out[...] = (rank < CAP).astype(jnp.int32)
        dst_out[...] = e_col // E_PER
        slot_out[...] = ((e_col % E_PER) * CAP
                         + jnp.minimum(rank.astype(jnp.int32), CAP - 1))

        tot_row = jnp.sum(jnp.sum(onehot3, axis=0), axis=0,
                          keepdims=True)
        iota_k = lax.broadcasted_iota(jnp.int32, (E_PER, N_EXP), 0)
        iota_e2 = lax.broadcasted_iota(jnp.int32, (E_PER, N_EXP), 1)
        sel = (iota_e2 == E_PER * my + iota_k).astype(jnp.float32)
        n_pair = jnp.sum(sel * tot_row, axis=1, keepdims=True)
        n_out[...] = jnp.minimum(n_pair, float(CAP)).astype(jnp.int32)

        iota_s = lax.broadcasted_iota(jnp.int32, (N_DEV, T_LOC), 0)
        iota_j = lax.broadcasted_iota(jnp.int32, (N_DEV, T_LOC), 1)
        gval = (iota_s * T_LOC + iota_j).astype(jnp.float32)
        m_up = (ii < jj).astype(jnp.float32)
        l32a = lax.broadcasted_iota(jnp.int32, (N_DEV, N_DEV), 0)
        l32b = lax.broadcasted_iota(jnp.int32, (N_DEV, N_DEV), 1)
        l32 = (l32b < l32a).astype(jnp.float32)
        iota_r = lax.broadcasted_iota(jnp.int32, (N_DEV, T_LOC, CAP), 2)
        tds, trs = [], []
        for k in range(E_PER):
            maskk = (ridx_all == E_PER * my + k).astype(jnp.float32)
            cntk = jnp.sum(maskk, axis=1, keepdims=True)
            pk = jnp.dot(l32, cntk, preferred_element_type=jnp.float32)
            lexcl = jnp.dot(maskk, m_up,
                            preferred_element_type=jnp.float32)
            rankk = (pk + lexcl).astype(jnp.int32)
            validk = maskk * (rankk < CAP).astype(jnp.float32)
            oh_r = (rankk[:, :, None] == iota_r).astype(jnp.float32)
            contrib = (validk * gval)[:, :, None] * oh_r
            tk = jnp.sum(jnp.sum(contrib, axis=0), axis=0,
                         keepdims=True).astype(jnp.int32)
            tds.append(tk // T_LOC)
            trs.append(tk % T_LOC)
        td_out[...] = jnp.concatenate(tds, axis=0)
        tr_out[...] = jnp.concatenate(trs, axis=0)

        for r in sends:
            r.wait_send()

    i32 = jnp.int32
    return pl.pallas_call(
        body,
        out_shape=[
            jax.ShapeDtypeStruct((T_LOC, 1), i32),
            jax.ShapeDtypeStruct((T_LOC, 1), i32),
            jax.ShapeDtypeStruct((T_LOC, 1), i32),
            jax.ShapeDtypeStruct((E_PER, 1), i32),
            jax.ShapeDtypeStruct((E_PER, CAP), i32),
            jax.ShapeDtypeStruct((E_PER, CAP), i32),
        ],
        in_specs=[pl.BlockSpec(memory_space=pltpu.VMEM)] * 2,
        out_specs=[pl.BlockSpec(memory_space=pltpu.VMEM)] * 6,
        scratch_shapes=[
            pltpu.VMEM((N_DEV, T_LOC), jnp.int32),
            pltpu.SemaphoreType.DMA((N_DEV - 1,)),
            pltpu.SemaphoreType.DMA((N_DEV,)),
        ],
        compiler_params=pltpu.CompilerParams(collective_id=0),
    )(ridx_row, ridx_col)


def kernel(x, router_W, route_idx, expert_W):
    del router_W
    x = x.astype(jnp.bfloat16).reshape(T_LOC, 1, D)
    ew = expert_W.astype(jnp.bfloat16)

    keep_t, dst_t, slot_t, n_t, td_t, tr_t = _route_tables(
        route_idx.reshape(1, T_LOC), route_idx.reshape(T_LOC, 1))
    keep_t = keep_t.reshape(T_LOC)
    dst_t = dst_t.reshape(T_LOC)
    slot_t = slot_t.reshape(T_LOC)
    n_t = n_t.reshape(E_PER)

    def body(keep_ref, dst_ref, slot_ref, n_ref, td_ref, tr_ref,
             x_ref, ew_ref, out_ref,
             x_stage, res_ref, out_stage, send_x, recv_x, send_r, recv_r):
        my = lax.axis_index("i")
        _barrier(my)
        out_stage[...] = jnp.zeros((T_LOC, 1, H), jnp.bfloat16)

        def disp(i, c):
            @pl.when((keep_ref[i] != 0) & (dst_ref[i] != my))
            def _():
                pltpu.make_async_remote_copy(
                    src_ref=x_ref.at[i],
                    dst_ref=x_stage.at[slot_ref[i]],
                    send_sem=send_x.at[i],
                    recv_sem=recv_x.at[slot_ref[i]],
                    device_id=(dst_ref[i],),
                    device_id_type=pl.DeviceIdType.MESH,
                ).start()

            @pl.when((keep_ref[i] != 0) & (dst_ref[i] == my))
            def _():
                x_stage[slot_ref[i]] = x_ref[i]
            return c
        lax.fori_loop(0, T_LOC, disp, 0)

        for k in range(E_PER):
            def wrecv(r, c, k=k):
                s = k * CAP + r

                @pl.when(td_ref[k, r] != my)
                def _():
                    pltpu.make_async_remote_copy(
                        src_ref=x_ref.at[0],
                        dst_ref=x_stage.at[s],
                        send_sem=send_x.at[0],
                        recv_sem=recv_x.at[s],
                        device_id=(my,),
                        device_id_type=pl.DeviceIdType.MESH,
                    ).wait_recv()
                return c
            lax.fori_loop(0, n_ref[k], wrecv, 0)

        iota = lax.broadcasted_iota(jnp.int32, (CAP, 1), 0)
        for k in range(E_PER):
            mask = (iota < n_ref[k]).astype(jnp.bfloat16)
            xs = x_stage[k * CAP:(k + 1) * CAP].reshape(CAP, D) * mask
            y = jnp.dot(xs, ew_ref[k], preferred_element_type=jnp.float32)
            res_ref[k * CAP:(k + 1) * CAP] = (
                y.astype(jnp.bfloat16).reshape(CAP, 1, H))

        for k in range(E_PER):
            def ret(r, c, k=k):
                s = k * CAP + r

                @pl.when(td_ref[k, r] != my)
                def _():
                    pltpu.make_async_remote_copy(
                        src_ref=res_ref.at[s],
                        dst_ref=out_stage.at[tr_ref[k, r]],
                        send_sem=send_r.at[s],
                        recv_sem=recv_r.at[tr_ref[k, r]],
                        device_id=(td_ref[k, r],),
                        device_id_type=pl.DeviceIdType.MESH,
                    ).start()

                @pl.when(td_ref[k, r] == my)
                def _():
                    out_stage[tr_ref[k, r]] = res_ref[s]
                return c
            lax.fori_loop(0, n_ref[k], ret, 0)

        def coll(i, c):
            @pl.when((keep_ref[i] != 0) & (dst_ref[i] != my))
            def _():
                pltpu.make_async_remote_copy(
                    src_ref=res_ref.at[0],
                    dst_ref=out_stage.at[i],
                    send_sem=send_r.at[0],
                    recv_sem=recv_r.at[i],
                    device_id=(my,),
                    device_id_type=pl.DeviceIdType.MESH,
                ).wait_recv()
                pltpu.make_async_remote_copy(
                    src_ref=x_ref.at[i],
                    dst_ref=x_stage.at[0],
                    send_sem=send_x.at[i],
                    recv_sem=recv_x.at[0],
                    device_id=(my,),
                    device_id_type=pl.DeviceIdType.MESH,
                ).wait_send()
            return c
        lax.fori_loop(0, T_LOC, coll, 0)
        for k in range(E_PER):
            def dret(r, c, k=k):
                s = k * CAP + r

                @pl.when(td_ref[k, r] != my)
                def _():
                    pltpu.make_async_remote_copy(
                        src_ref=res_ref.at[s],
                        dst_ref=out_stage.at[0],
                        send_sem=send_r.at[s],
                        recv_sem=recv_r.at[0],
                        device_id=(my,),
                        device_id_type=pl.DeviceIdType.MESH,
                    ).wait_send()
                return c
            lax.fori_loop(0, n_ref[k], dret, 0)

        out_ref[...] = out_stage[...].astype(jnp.float32)

    smem = pl.BlockSpec(memory_space=pltpu.SMEM)
    vmem = pl.BlockSpec(memory_space=pltpu.VMEM)
    out = pl.pallas_call(
        body,
        out_shape=jax.ShapeDtypeStruct((T_LOC, 1, H), jnp.float32),
        in_specs=[smem, smem, smem, smem, smem, smem, vmem, vmem],
        out_specs=vmem,
        scratch_shapes=[
            pltpu.VMEM((E_PER * CAP, 1, D), jnp.bfloat16),
            pltpu.VMEM((E_PER * CAP, 1, H), jnp.bfloat16),
            pltpu.VMEM((T_LOC, 1, H), jnp.bfloat16),
            pltpu.SemaphoreType.DMA((T_LOC,)),
            pltpu.SemaphoreType.DMA((E_PER * CAP,)),
            pltpu.SemaphoreType.DMA((E_PER * CAP,)),
            pltpu.SemaphoreType.DMA((T_LOC,)),
        ],
        compiler_params=pltpu.CompilerParams(collective_id=1),
    )(keep_t, dst_t, slot_t, n_t, td_t, tr_t, x, ew)
    return out.reshape(T_LOC, H)
